# baseline (speedup 1.0000x reference)
import functools

import jax
import jax.numpy as jnp
from jax import lax
from jax.experimental import pallas as pl
from jax.experimental.pallas import tpu as pltpu
from jax.experimental.pallas import tpu_sc as plsc

BATCH = 16384
DIM = 64

_NC = 2
_NS = 16
_NW = _NC * _NS
_BPW = BATCH // _NW


def _sc_gather_body(uids, iids, gu_t, gi_t, mu_t, mi_t,
                    out_gu, out_gi, out_mu, out_mi,
                    uidx_v, iidx_v, buf_a, buf_b, sem_a, sem_b):
    wid = lax.axis_index("s") * _NC + lax.axis_index("c")
    base = wid * _BPW
    pltpu.sync_copy(uids.at[pl.ds(base, _BPW)], uidx_v)
    pltpu.sync_copy(iids.at[pl.ds(base, _BPW)], iidx_v)
    c_a = pltpu.async_copy(gu_t.at[uidx_v], buf_a, sem_a)
    c_b = pltpu.async_copy(gi_t.at[iidx_v], buf_b, sem_b)
    c_a.wait()
    pltpu.sync_copy(buf_a, out_gu.at[pl.ds(base, _BPW)])
    c_a2 = pltpu.async_copy(mu_t.at[uidx_v], buf_a, sem_a)
    c_b.wait()
    pltpu.sync_copy(buf_b, out_gi.at[pl.ds(base, _BPW)])
    c_b2 = pltpu.async_copy(mi_t.at[iidx_v], buf_b, sem_b)
    c_a2.wait()
    pltpu.sync_copy(buf_a, out_mu.at[pl.ds(base, _BPW)])
    c_b2.wait()
    pltpu.sync_copy(buf_b, out_mi.at[pl.ds(base, _BPW)])


@functools.cache
def _make_sc_gather():
    return functools.partial(
        pl.kernel,
        out_type=[jax.ShapeDtypeStruct((BATCH, DIM), jnp.float32)] * 4,
        mesh=plsc.VectorSubcoreMesh(core_axis_name="c", subcore_axis_name="s"),
        compiler_params=pltpu.CompilerParams(use_tc_tiling_on_sc=False),
        scratch_types=[
            pltpu.VMEM((_BPW,), jnp.int32),
            pltpu.VMEM((_BPW,), jnp.int32),
            pltpu.VMEM((_BPW, DIM), jnp.float32),
            pltpu.VMEM((_BPW, DIM), jnp.float32),
            pltpu.SemaphoreType.DMA,
            pltpu.SemaphoreType.DMA,
        ],
    )(_sc_gather_body)


_BLK = 2048


def _tc_mlp_body(gu, gi, mu, mi,
                 w0u, w0i, b0, s0, t0,
                 w1, b1, s1, t1,
                 w2, b2, s2, t2,
                 wg, wx, bo, out):
    x = mu[...] @ w0u[...] + mi[...] @ w0i[...] + b0[...]
    x = jnp.maximum(x, 0.0) * s0[...] + t0[...]
    x = x @ w1[...] + b1[...]
    x = jnp.maximum(x, 0.0) * s1[...] + t1[...]
    x = x @ w2[...] + b2[...]
    x = jnp.maximum(x, 0.0) * s2[...] + t2[...]
    g = gu[...] * gi[...]
    logit = (jnp.sum(g * wg[...], axis=1, keepdims=True)
             + jnp.sum(x * wx[...], axis=1, keepdims=True) + bo[...])
    out[...] = jax.nn.sigmoid(logit)


def _tc_mlp(gu, gi, mu, mi, params):
    n_blk = BATCH // _BLK
    data_spec = pl.BlockSpec((_BLK, DIM), lambda i: (i, 0))

    def full(a):
        return pl.BlockSpec(a.shape, lambda i: (0,) * a.ndim)

    in_specs = [data_spec] * 4 + [full(p) for p in params]
    return pl.pallas_call(
        _tc_mlp_body,
        grid=(n_blk,),
        in_specs=in_specs,
        out_specs=pl.BlockSpec((_BLK, 1), lambda i: (i, 0)),
        out_shape=jax.ShapeDtypeStruct((BATCH, 1), jnp.float32),
    )(gu, gi, mu, mi, *params)


def kernel(inputs, gmf_user_table, gmf_item_table, mlp_user_table, mlp_item_table,
           W0, b0, g0, be0, m0, v0,
           W1, b1, g1, be1, m1, v1,
           W2, b2, g2, be2, m2, v2,
           Wout, bout):
    uids = inputs[:, 0].astype(jnp.int32)
    iids = inputs[:, 1].astype(jnp.int32)

    gu, gi, mu, mi = _make_sc_gather()(
        uids, iids, gmf_user_table, gmf_item_table,
        mlp_user_table, mlp_item_table)

    def fold(g, be, m, v):
        s = g / jnp.sqrt(v + 1e-3)
        return s, be - m * s

    s0, t0 = fold(g0, be0, m0, v0)
    s1, t1 = fold(g1, be1, m1, v1)
    s2, t2 = fold(g2, be2, m2, v2)

    def row(a):
        return a.reshape(1, -1)

    params = [
        W0[:DIM], W0[DIM:], row(b0), row(s0), row(t0),
        W1, row(b1), row(s1), row(t1),
        W2, row(b2), row(s2), row(t2),
        row(Wout[:DIM, 0]), row(Wout[DIM:, 0]), row(bout),
    ]
    out = _tc_mlp(gu, gi, mu, mi, params)
    return jnp.squeeze(out, axis=1)


# speedup vs baseline: 1.5054x; 1.5054x over previous
import functools

import jax
import jax.numpy as jnp
from jax import lax
from jax.experimental import pallas as pl
from jax.experimental.pallas import tpu as pltpu
from jax.experimental.pallas import tpu_sc as plsc

BATCH = 16384
DIM = 64

_NC = 2
_NS = 16
_NW = _NC * _NS
_BPW = BATCH // _NW


_CHUNK = 128


def _sc_gather_body(uids, iids, gu_t, gi_t, mu_t, mi_t,
                    out_gu, out_gi, out_mu, out_mi,
                    uid_vm, iid_vm, bufs, sems):
    wid = lax.axis_index("s") * _NC + lax.axis_index("c")
    base = wid * _BPW
    pltpu.sync_copy(uids.at[pl.ds(base, _BPW)], uid_vm)
    pltpu.sync_copy(iids.at[pl.ds(base, _BPW)], iid_vm)
    uid_sm, iid_sm = uid_vm, iid_vm
    tables = (gu_t, gi_t, mu_t, mi_t)
    outs = (out_gu, out_gi, out_mu, out_mi)
    idxs = (uid_sm, iid_sm, uid_sm, iid_sm)

    for c in range(_BPW // _CHUNK):
        def issue(g, _):
            uvec = uid_sm[pl.ds(c * _CHUNK + g * 16, 16)]
            ivec = iid_sm[pl.ds(c * _CHUNK + g * 16, 16)]
            for j in range(16):
                k = g * 16 + j
                rows = (uvec[j], ivec[j], uvec[j], ivec[j])
                for t in range(4):
                    pltpu.make_async_copy(
                        tables[t].at[pl.ds(rows[t], 1)],
                        bufs[t].at[pl.ds(k, 1)], sems[t]).start()
            return 0

        lax.fori_loop(0, _CHUNK // 16, issue, 0)
        for t in range(4):
            pltpu.make_async_copy(
                tables[t].at[pl.ds(0, _CHUNK)], bufs[t], sems[t]).wait()
            pltpu.sync_copy(bufs[t], outs[t].at[pl.ds(base + c * _CHUNK, _CHUNK)])


@functools.cache
def _make_sc_gather():
    def body(uids, iids, gu_t, gi_t, mu_t, mi_t,
             out_gu, out_gi, out_mu, out_mi,
             uid_vm, iid_vm, b0, b1, b2, b3, s0, s1, s2, s3):
        _sc_gather_body(uids, iids, gu_t, gi_t, mu_t, mi_t,
                        out_gu, out_gi, out_mu, out_mi,
                        uid_vm, iid_vm,
                        (b0, b1, b2, b3), (s0, s1, s2, s3))

    return functools.partial(
        pl.kernel,
        out_type=[jax.ShapeDtypeStruct((BATCH, DIM), jnp.float32)] * 4,
        mesh=plsc.VectorSubcoreMesh(core_axis_name="c", subcore_axis_name="s"),
        scratch_types=[
            pltpu.VMEM((_BPW,), jnp.int32),
            pltpu.VMEM((_BPW,), jnp.int32),
        ] + [pltpu.VMEM((_CHUNK, DIM), jnp.float32)] * 4
          + [pltpu.SemaphoreType.DMA] * 4,
    )(body)


_BLK = 2048


def _tc_mlp_body(gu, gi, mu, mi,
                 w0u, w0i, b0, s0, t0,
                 w1, b1, s1, t1,
                 w2, b2, s2, t2,
                 wg, wx, bo, out):
    x = mu[...] @ w0u[...] + mi[...] @ w0i[...] + b0[...]
    x = jnp.maximum(x, 0.0) * s0[...] + t0[...]
    x = x @ w1[...] + b1[...]
    x = jnp.maximum(x, 0.0) * s1[...] + t1[...]
    x = x @ w2[...] + b2[...]
    x = jnp.maximum(x, 0.0) * s2[...] + t2[...]
    g = gu[...] * gi[...]
    logit = (jnp.sum(g * wg[...], axis=1, keepdims=True)
             + jnp.sum(x * wx[...], axis=1, keepdims=True) + bo[...])
    out[...] = jax.nn.sigmoid(logit)


def _tc_mlp(gu, gi, mu, mi, params):
    n_blk = BATCH // _BLK
    data_spec = pl.BlockSpec((_BLK, DIM), lambda i: (i, 0))

    def full(a):
        return pl.BlockSpec(a.shape, lambda i: (0,) * a.ndim)

    in_specs = [data_spec] * 4 + [full(p) for p in params]
    return pl.pallas_call(
        _tc_mlp_body,
        grid=(n_blk,),
        in_specs=in_specs,
        out_specs=pl.BlockSpec((_BLK, 1), lambda i: (i, 0)),
        out_shape=jax.ShapeDtypeStruct((BATCH, 1), jnp.float32),
    )(gu, gi, mu, mi, *params)


def kernel(inputs, gmf_user_table, gmf_item_table, mlp_user_table, mlp_item_table,
           W0, b0, g0, be0, m0, v0,
           W1, b1, g1, be1, m1, v1,
           W2, b2, g2, be2, m2, v2,
           Wout, bout):
    uids = inputs[:, 0].astype(jnp.int32)
    iids = inputs[:, 1].astype(jnp.int32)

    gu, gi, mu, mi = _make_sc_gather()(
        uids, iids, gmf_user_table, gmf_item_table,
        mlp_user_table, mlp_item_table)

    def fold(g, be, m, v):
        s = g / jnp.sqrt(v + 1e-3)
        return s, be - m * s

    s0, t0 = fold(g0, be0, m0, v0)
    s1, t1 = fold(g1, be1, m1, v1)
    s2, t2 = fold(g2, be2, m2, v2)

    def row(a):
        return a.reshape(1, -1)

    params = [
        W0[:DIM], W0[DIM:], row(b0), row(s0), row(t0),
        W1, row(b1), row(s1), row(t1),
        W2, row(b2), row(s2), row(t2),
        row(Wout[:DIM, 0]), row(Wout[DIM:, 0]), row(bout),
    ]
    out = _tc_mlp(gu, gi, mu, mi, params)
    return jnp.squeeze(out, axis=1)
